# baseline (speedup 1.0000x reference)
import functools

import jax
import jax.numpy as jnp
from jax import lax
from jax.experimental import pallas as pl
from jax.experimental.pallas import tpu as pltpu
from jax.experimental.pallas import tpu_sc as plsc

N = 10000
G = 64
H = 128

NC = 2
NS = 16
NW = NC * NS
SEGS_PER_W = G // NW
CHUNK = 256
NVREG = H // 16
NPAD = N + 16
SEARCH_STEPS = 14


_sc_mesh = plsc.VectorSubcoreMesh(core_axis_name="c", subcore_axis_name="s")


@functools.partial(
    pl.kernel,
    mesh=_sc_mesh,
    out_type=jax.ShapeDtypeStruct((G * H,), jnp.float32),
    scratch_types=[
        pltpu.VMEM((NPAD,), jnp.int32),
        pltpu.VMEM((CHUNK * H,), jnp.float32),
        pltpu.VMEM((SEGS_PER_W * H,), jnp.float32)
    ],
)
def _seg_mean_sc(x_hbm, batch_hbm, out_hbm, ids_v, buf, outbuf):
    wid = lax.axis_index("s") * NC + lax.axis_index("c")
    pltpu.sync_copy(batch_hbm, ids_v)

    def lower_bound(g):
        lo, hi = jnp.int32(0), jnp.int32(N)
        for _ in range(SEARCH_STEPS):
            mid = (lo + hi) >> 1
            v = ids_v[pl.ds(mid, 16)][0]
            cond = v < g
            lo = jnp.where(cond, mid + 1, lo)
            hi = jnp.where(cond, hi, mid)
        return lo

    for si in range(SEGS_PER_W):
        g = wid * SEGS_PER_W + si
        s = lower_bound(g)
        e = lower_bound(g + 1)
        n = e - s
        nch = (n + (CHUNK - 1)) // CHUNK

        def chunk_body(ci, accs):
            rstart = jnp.minimum(s + ci * CHUNK, N - CHUNK)
            pltpu.sync_copy(x_hbm.at[pl.ds(rstart * H, CHUNK * H)], buf)
            lo = jnp.maximum(s, rstart) - rstart
            hi = jnp.minimum(e, rstart + CHUNK) - rstart

            def row_body(i, a):
                return tuple(a[j] + buf[pl.ds(i * H + 16 * j, 16)]
                             for j in range(NVREG))

            return lax.fori_loop(lo, hi, row_body, accs)

        accs0 = tuple(jnp.zeros((16,), jnp.float32) for _ in range(NVREG))
        accs = lax.fori_loop(0, nch, chunk_body, accs0)
        nv = jnp.full((16,), jnp.maximum(n, 1), jnp.float32)
        inv = 1.0 / nv
        for j in range(NVREG):
            outbuf[pl.ds(si * H + 16 * j, 16)] = accs[j] * inv

    pltpu.sync_copy(
        outbuf, out_hbm.at[pl.ds(wid * (SEGS_PER_W * H), SEGS_PER_W * H)])


def _mlp_body(mean_ref, W1_ref, b1_ref, gamma_ref, beta_ref, W2_ref, b2_ref,
              out_ref):
    h = lax.dot_general(mean_ref[...], W1_ref[...],
                        dimension_numbers=(((1,), (0,)), ((), ())),
                        preferred_element_type=jnp.float32)
    h = h + b1_ref[...]
    mu = jnp.mean(h, axis=0, keepdims=True)
    var = jnp.mean((h - mu) ** 2, axis=0, keepdims=True)
    h = (h - mu) * lax.rsqrt(var + 1e-5) * gamma_ref[...] + beta_ref[...]
    h = jnp.maximum(h, 0.0)
    out = lax.dot_general(h, W2_ref[...],
                          dimension_numbers=(((1,), (0,)), ((), ())),
                          preferred_element_type=jnp.float32)
    out_ref[...] = out + b2_ref[...]


def _mlp(mean, W1, b1, gamma, beta, W2, b2):
    return pl.pallas_call(
        _mlp_body,
        out_shape=jax.ShapeDtypeStruct((G, H), jnp.float32),
    )(mean, W1, b1.reshape(1, H), gamma.reshape(1, H), beta.reshape(1, H),
      W2, b2.reshape(1, H))


def kernel(x, edge_index, edge_attr, u, batch, W1, b1, gamma, beta, W2, b2):
    del edge_index, edge_attr, u
    batch_p = jnp.pad(batch.astype(jnp.int32), (0, NPAD - N),
                      constant_values=G)
    mean = _seg_mean_sc(x.reshape(N * H), batch_p).reshape(G, H)
    return _mlp(mean, W1, b1, gamma, beta, W2, b2)


# speedup vs baseline: 1.0461x; 1.0461x over previous
import functools

import jax
import jax.numpy as jnp
from jax import lax
from jax.experimental import pallas as pl
from jax.experimental.pallas import tpu as pltpu
from jax.experimental.pallas import tpu_sc as plsc

N = 10000
G = 64
H = 128
G_SC = 32

NC = 2
NS = 16
NW = NC * NS
CHUNK = 256
NVREG = H // 16
NPAD = N + 16
SEARCH_STEPS = 14


_sc_mesh = plsc.VectorSubcoreMesh(core_axis_name="c", subcore_axis_name="s")


@functools.partial(
    pl.kernel,
    mesh=_sc_mesh,
    out_type=jax.ShapeDtypeStruct((G_SC * H,), jnp.float32),
    scratch_types=[
        pltpu.VMEM((NPAD,), jnp.int32),
        pltpu.VMEM((CHUNK * H,), jnp.float32),
        pltpu.VMEM((H,), jnp.float32),
    ],
)
def _seg_mean_sc(x_hbm, batch_hbm, out_hbm, ids_v, buf, outbuf):
    wid = lax.axis_index("s") * NC + lax.axis_index("c")
    pltpu.sync_copy(batch_hbm, ids_v)

    def lower_bound(g):
        lo, hi = jnp.int32(0), jnp.int32(N)
        for _ in range(SEARCH_STEPS):
            mid = (lo + hi) >> 1
            v = ids_v[pl.ds(mid, 16)][0]
            cond = v < g
            lo = jnp.where(cond, mid + 1, lo)
            hi = jnp.where(cond, hi, mid)
        return lo

    g = wid
    s = lower_bound(g)
    e = lower_bound(g + 1)
    n = e - s
    nch = (n + (CHUNK - 1)) // CHUNK

    def chunk_body(ci, accs):
        rstart = jnp.minimum(s + ci * CHUNK, N - CHUNK)
        pltpu.sync_copy(x_hbm.at[pl.ds(rstart * H, CHUNK * H)], buf)
        lo = jnp.maximum(s, rstart) - rstart
        hi = jnp.minimum(e, rstart + CHUNK) - rstart

        def row_body(i, a):
            return tuple(a[j] + buf[pl.ds(i * H + 16 * j, 16)]
                         for j in range(NVREG))

        return lax.fori_loop(lo, hi, row_body, accs)

    accs0 = tuple(jnp.zeros((16,), jnp.float32) for _ in range(NVREG))
    accs = lax.fori_loop(0, nch, chunk_body, accs0)
    nv = jnp.full((16,), jnp.maximum(n, 1), jnp.float32)
    inv = 1.0 / nv
    for j in range(NVREG):
        outbuf[pl.ds(16 * j, 16)] = accs[j] * inv

    pltpu.sync_copy(outbuf, out_hbm.at[pl.ds(wid * H, H)])


BLK = 1000
NB = N // BLK


def _onehot_body(batch_ref, x_ref, out_ref, acc_ref, cnt_ref):
    k = pl.program_id(0)

    @pl.when(k == 0)
    def _init():
        acc_ref[...] = jnp.zeros_like(acc_ref)
        cnt_ref[...] = jnp.zeros_like(cnt_ref)

    ids = batch_ref[0, 0, :]
    oh = (ids[:, None] == lax.broadcasted_iota(jnp.int32, (BLK, G), 1))
    oh = oh.astype(jnp.float32)
    xb = x_ref[...]
    acc_ref[...] += lax.dot_general(
        oh, xb, dimension_numbers=(((0,), (0,)), ((), ())),
        preferred_element_type=jnp.float32)
    cnt_ref[...] += lax.dot_general(
        oh, jnp.ones((BLK, H), jnp.float32),
        dimension_numbers=(((0,), (0,)), ((), ())),
        preferred_element_type=jnp.float32)

    @pl.when(k == NB - 1)
    def _finish():
        out_ref[...] = acc_ref[...] / jnp.maximum(cnt_ref[...], 1.0)


def _onehot_mean(batch3, x):
    return pl.pallas_call(
        _onehot_body,
        grid=(NB,),
        in_specs=[
            pl.BlockSpec((1, 1, BLK), lambda k: (k, 0, 0)),
            pl.BlockSpec((BLK, H), lambda k: (k, 0)),
        ],
        out_specs=pl.BlockSpec((G, H), lambda k: (0, 0)),
        out_shape=jax.ShapeDtypeStruct((G, H), jnp.float32),
        scratch_shapes=[
            pltpu.VMEM((G, H), jnp.float32),
            pltpu.VMEM((G, H), jnp.float32),
        ],
    )(batch3, x)


def _mlp_body(msc_ref, mtc_ref, W1_ref, b1_ref, gamma_ref, beta_ref,
              W2_ref, b2_ref, out_ref):
    mean = jnp.concatenate([msc_ref[...], mtc_ref[G_SC:G, :]], axis=0)
    h = lax.dot_general(mean, W1_ref[...],
                        dimension_numbers=(((1,), (0,)), ((), ())),
                        preferred_element_type=jnp.float32)
    h = h + b1_ref[...]
    mu = jnp.mean(h, axis=0, keepdims=True)
    var = jnp.mean((h - mu) ** 2, axis=0, keepdims=True)
    h = (h - mu) * lax.rsqrt(var + 1e-5) * gamma_ref[...] + beta_ref[...]
    h = jnp.maximum(h, 0.0)
    out = lax.dot_general(h, W2_ref[...],
                          dimension_numbers=(((1,), (0,)), ((), ())),
                          preferred_element_type=jnp.float32)
    out_ref[...] = out + b2_ref[...]


def _mlp(mean_sc, mean_tc, W1, b1, gamma, beta, W2, b2):
    return pl.pallas_call(
        _mlp_body,
        out_shape=jax.ShapeDtypeStruct((G, H), jnp.float32),
    )(mean_sc, mean_tc, W1, b1.reshape(1, H), gamma.reshape(1, H),
      beta.reshape(1, H), W2, b2.reshape(1, H))


def kernel(x, edge_index, edge_attr, u, batch, W1, b1, gamma, beta, W2, b2):
    del edge_index, edge_attr, u
    batch_i = batch.astype(jnp.int32)
    batch_p = jnp.pad(batch_i, (0, NPAD - N), constant_values=G)
    batch3 = batch_i.reshape(NB, 1, BLK)
    mean_sc = _seg_mean_sc(x.reshape(N * H), batch_p).reshape(G_SC, H)
    mean_tc = _onehot_mean(batch3, x)
    return _mlp(mean_sc, mean_tc, W1, b1, gamma, beta, W2, b2)
